# baseline (speedup 1.0000x reference)
import functools

import jax
import jax.numpy as jnp
from jax.experimental import pallas as pl
from jax.experimental.pallas import tpu as pltpu

N_NODES = 10000
N_GRAPHS = 8
K = 16
BN_EPS = 1e-5

NPAD = 10240
BQ = 512
CT = 1536
NEXT = NPAD + CT
NBLK = NPAD // BQ
BIGF = float(NEXT)
F32 = jnp.float32


def _dot(a, b, dims):
    return jax.lax.dot_general(a, b, (dims, ((), ())),
                               preferred_element_type=F32)


def _prep_body(x_ref, br_ref, bc_ref, h0_ref):
    x = x_ref[...]
    gid = jax.lax.broadcasted_iota(jnp.int32, (N_GRAPHS, NEXT), 0)
    onehot = (gid == br_ref[...]).astype(F32)
    gmsum = _dot(onehot, x, ((1,), (0,)))
    cnt = jnp.sum(onehot, axis=1, keepdims=True)
    gm = gmsum / jnp.maximum(cnt, 1.0)
    gidT = jax.lax.broadcasted_iota(jnp.int32, (NEXT, N_GRAPHS), 1)
    onehotT = (gidT == bc_ref[...]).astype(F32)
    gmb = _dot(onehotT, gm, ((1,), (0,)))
    h0_ref[...] = jnp.concatenate([x, gmb], axis=1)


def _elu(a):
    return jnp.where(a > 0, a, jnp.exp(a) - 1.0)


def _dense_body(h_ref, w1, b1, w2, b2, w3, b3, sw, sb, hw, hb,
                t_ref, a_ref, bside_ref, hm_ref):
    h = h_ref[...]
    a = _elu(_dot(h, w1[...], ((1,), (0,))) + b1[...])
    a = _elu(_dot(a, w2[...], ((1,), (0,))) + b2[...])
    t = jnp.tanh(_dot(a, w3[...], ((1,), (0,))) + b3[...])
    t_ref[...] = t
    s = _dot(t, sw[...], ((1,), (0,))) + sb[...]
    sq = jnp.sum(s * s, axis=1, keepdims=True)
    one = jnp.ones_like(sq)
    zero3 = jnp.zeros((NEXT, 3), F32)
    a_ref[...] = jnp.concatenate([-2.0 * s, sq, one, zero3], axis=1)
    bside_ref[...] = jnp.concatenate([s, one, sq, zero3], axis=1)
    hm_ref[...] = _dot(t, hw[...], ((1,), (0,))) + hb[...]


def _gravnet_body(lo_ref, nch_ref, aq_ref, bq_ref, tq_ref, bfull_ref,
                  bcand_ref, hm_ref, o1, o2a, o2b, bias, out_ref, d_scr):
    b = pl.program_id(0)
    lo = lo_ref[b]
    nch = nch_ref[b]
    aq = aq_ref[...]
    batq = bq_ref[...]

    inf = jnp.full((BQ, 1), jnp.inf, F32)
    bigc = jnp.full((BQ, 1), BIGF, F32)

    def colids(base):
        return (jax.lax.broadcasted_iota(jnp.int32, (BQ, CT), 1)
                + base).astype(F32)

    def chunk_min(d, colid, m, c):
        cm = jnp.min(d, axis=1, keepdims=True)
        ci = jnp.min(jnp.where(d == cm, colid, BIGF), axis=1, keepdims=True)
        take = cm < m
        return jnp.where(take, cm, m), jnp.where(take, ci, c)

    def abody(j, carry):
        m, c = carry
        base = pl.multiple_of(lo + j * CT, 128)
        bc = _dot(aq, bfull_ref[pl.ds(base, CT), :], ((1,), (1,)))
        same = batq == bcand_ref[0:1, pl.ds(base, CT)]
        d = jnp.where(same, bc, jnp.inf)
        d_scr[:, pl.ds(base, CT)] = d
        return chunk_min(d, colids(base), m, c)

    m0, c0 = jax.lax.fori_loop(0, nch, abody, (inf, bigc))

    def tbody(_, carry):
        m, c, mean, mx = carry
        w = jnp.exp(-10.0 * m)

        def jbody(j, icarry):
            m2, c2, acc = icarry
            base = pl.multiple_of(lo + j * CT, 128)
            chunk = d_scr[:, pl.ds(base, CT)]
            colid = colids(base)
            hit = colid == c
            acc = acc + _dot(hit.astype(F32), hm_ref[pl.ds(base, CT), :],
                             ((1,), (0,)))
            chunk = jnp.where(hit, jnp.inf, chunk)
            d_scr[:, pl.ds(base, CT)] = chunk
            m2, c2 = chunk_min(chunk, colid, m2, c2)
            return m2, c2, acc

        m2, c2, acc = jax.lax.fori_loop(
            0, nch, jbody, (inf, bigc, jnp.zeros((BQ, 16), F32)))
        msg = w * acc
        return m2, c2, mean + msg, jnp.maximum(mx, msg)

    _, _, mean, mx = jax.lax.fori_loop(
        0, K, tbody,
        (m0, c0, jnp.zeros((BQ, 16), F32), jnp.full((BQ, 16), -jnp.inf, F32)))

    out_ref[...] = (_dot(tq_ref[...], o1[...], ((1,), (0,)))
                    + _dot(mean * (1.0 / K), o2a[...], ((1,), (0,)))
                    + _dot(mx, o2b[...], ((1,), (0,)))
                    + bias[...])


def _final_body(hc_ref, br_ref, c1w, c1b, c2w, c2b, c3w, c3b, out_ref):
    gid = jax.lax.broadcasted_iota(jnp.int32, (N_GRAPHS, NPAD), 0)
    onehot = (gid == br_ref[...]).astype(F32)
    xs = _dot(onehot, hc_ref[...], ((1,), (0,)))
    cnt = jnp.sum(onehot, axis=1, keepdims=True)
    xp = xs / jnp.maximum(cnt, 1.0)
    z = jnp.maximum(_dot(xp, c1w[...], ((1,), (0,))) + c1b[...], 0.0)
    z = jnp.maximum(_dot(z, c2w[...], ((1,), (0,))) + c2b[...], 0.0)
    out_ref[...] = _dot(z, c3w[...], ((1,), (0,))) + c3b[...]


def _whole(shape):
    n = len(shape)
    return pl.BlockSpec(shape, lambda b: (0,) * n)


def _run_prep(xp, brow, bcol):
    return pl.pallas_call(
        _prep_body,
        out_shape=jax.ShapeDtypeStruct((NEXT, 8), F32),
    )(xp, brow, bcol)


def _run_dense(h, p, i):
    fin = h.shape[1]
    outs = (jax.ShapeDtypeStruct((NEXT, 16), F32),
            jax.ShapeDtypeStruct((NEXT, 8), F32),
            jax.ShapeDtypeStruct((NEXT, 8), F32),
            jax.ShapeDtypeStruct((NEXT, 16), F32))
    r2 = lambda v: v.reshape(1, -1)
    return pl.pallas_call(_dense_body, out_shape=outs)(
        h,
        p['ft%d_1_W' % i], r2(p['ft%d_1_b' % i]),
        p['ft%d_2_W' % i], r2(p['ft%d_2_b' % i]),
        p['ft%d_3_W' % i], r2(p['ft%d_3_b' % i]),
        p['gn%d_s_W' % i], r2(p['gn%d_s_b' % i]),
        p['gn%d_h_W' % i], r2(p['gn%d_h_b' % i]))


def _run_gravnet(lo, nch, A, Bs, t, hm, bcol, brow, o1, o2a, o2b, bias):
    grid = (NBLK,)
    in_specs = [
        pl.BlockSpec(memory_space=pltpu.SMEM),
        pl.BlockSpec(memory_space=pltpu.SMEM),
        pl.BlockSpec((BQ, 8), lambda b: (b, 0)),
        pl.BlockSpec((BQ, 1), lambda b: (b, 0)),
        pl.BlockSpec((BQ, 16), lambda b: (b, 0)),
        _whole((NEXT, 8)),
        _whole((1, NEXT)),
        _whole((NEXT, 16)),
        _whole((16, 16)), _whole((16, 16)), _whole((16, 16)),
        _whole((1, 16)),
    ]
    return pl.pallas_call(
        _gravnet_body,
        grid=grid,
        in_specs=in_specs,
        out_specs=pl.BlockSpec((BQ, 16), lambda b: (b, 0)),
        out_shape=jax.ShapeDtypeStruct((NPAD, 16), F32),
        scratch_shapes=[pltpu.VMEM((BQ, NEXT), F32)],
        compiler_params=pltpu.CompilerParams(
            dimension_semantics=("arbitrary",)),
    )(lo, nch, A, bcol, t, Bs, brow, hm, o1, o2a, o2b, bias)


def _run_final(hc, brow, p):
    r2 = lambda v: v.reshape(1, -1)
    return pl.pallas_call(
        _final_body,
        out_shape=jax.ShapeDtypeStruct((N_GRAPHS, 3), F32),
    )(hc, brow, p['c1_W'], r2(p['c1_b']), p['c2_W'], r2(p['c2_b']),
      p['c3_W'], r2(p['c3_b']))


@jax.jit
def kernel(x, edge_index, batch, params):
    del edge_index
    p = params
    pad = NEXT - N_NODES
    xp = jnp.pad(x, ((0, pad), (0, 0)))
    bpad = jnp.concatenate(
        [batch.astype(jnp.int32),
         jnp.full((pad,), N_GRAPHS, jnp.int32)])
    bcol = bpad[:, None]
    brow = bpad[None, :]

    offs = jnp.searchsorted(bpad, jnp.arange(N_GRAPHS + 2, dtype=jnp.int32),
                            side='left').astype(jnp.int32)
    qs = jnp.arange(NBLK, dtype=jnp.int32) * BQ
    g_lo = bpad[qs]
    g_hi = bpad[qs + BQ - 1]
    lo_raw = offs[g_lo]
    hi = offs[g_hi + 1]
    lo = (lo_raw // 128) * 128
    nch = (hi - lo + CT - 1) // CT

    h = _run_prep(xp, brow, bcol)
    feats = []
    for i in (1, 2, 3):
        t, A, Bs, hm = _run_dense(h, p, i)
        sc = p['bn%d_g' % i] / jnp.sqrt(1.0 + BN_EPS)
        o1 = p['gn%d_o1W' % i] * sc[None, :]
        o2 = p['gn%d_o2W' % i] * sc[None, :]
        bias = (p['gn%d_o2b' % i] * sc + p['bn%d_b' % i]).reshape(1, 16)
        hq = _run_gravnet(lo, nch, A, Bs, t, hm, bcol, brow,
                          o1, o2[:16], o2[16:], bias)
        feats.append(hq)
        h = jnp.pad(hq, ((0, CT), (0, 0)))
    hc = jnp.concatenate(feats, axis=1)
    return _run_final(hc, brow[:, :NPAD], p)


# speedup vs baseline: 1.1342x; 1.1342x over previous
import functools

import jax
import jax.numpy as jnp
from jax.experimental import pallas as pl
from jax.experimental.pallas import tpu as pltpu

N_NODES = 10000
N_GRAPHS = 8
K = 16
BN_EPS = 1e-5

NPAD = 10240
BQ = 256
CT = 1536
NEXT = NPAD + CT
NBLK = NPAD // BQ
BIGF = float(NEXT)
F32 = jnp.float32


def _dot(a, b, dims):
    return jax.lax.dot_general(a, b, (dims, ((), ())),
                               preferred_element_type=F32)


def _prep_body(x_ref, br_ref, bc_ref, h0_ref):
    x = x_ref[...]
    gid = jax.lax.broadcasted_iota(jnp.int32, (N_GRAPHS, NEXT), 0)
    onehot = (gid == br_ref[...]).astype(F32)
    gmsum = _dot(onehot, x, ((1,), (0,)))
    cnt = jnp.sum(onehot, axis=1, keepdims=True)
    gm = gmsum / jnp.maximum(cnt, 1.0)
    gidT = jax.lax.broadcasted_iota(jnp.int32, (NEXT, N_GRAPHS), 1)
    onehotT = (gidT == bc_ref[...]).astype(F32)
    gmb = _dot(onehotT, gm, ((1,), (0,)))
    h0_ref[...] = jnp.concatenate([x, gmb], axis=1)


def _elu(a):
    return jnp.where(a > 0, a, jnp.exp(a) - 1.0)


def _dense_body(h_ref, w1, b1, w2, b2, w3, b3, sw, sb, hw, hb,
                t_ref, a_ref, bside_ref, hm_ref):
    h = h_ref[...]
    a = _elu(_dot(h, w1[...], ((1,), (0,))) + b1[...])
    a = _elu(_dot(a, w2[...], ((1,), (0,))) + b2[...])
    t = jnp.tanh(_dot(a, w3[...], ((1,), (0,))) + b3[...])
    t_ref[...] = t
    s = _dot(t, sw[...], ((1,), (0,))) + sb[...]
    sq = jnp.sum(s * s, axis=1, keepdims=True)
    one = jnp.ones_like(sq)
    zero3 = jnp.zeros((NEXT, 3), F32)
    a_ref[...] = jnp.concatenate([-2.0 * s, sq, one, zero3], axis=1)
    bside_ref[...] = jnp.concatenate([s, one, sq, zero3], axis=1)
    hm_ref[...] = _dot(t, hw[...], ((1,), (0,))) + hb[...]


def _gravnet_body(lo_ref, nch_ref, aq_ref, bq_ref, tq_ref, bfull_ref,
                  bcand_ref, hm_ref, o1, o2a, o2b, bias, out_ref, d_scr):
    b = pl.program_id(0)
    lo = lo_ref[b]
    nch = nch_ref[b]
    aq = aq_ref[...]
    batq = bq_ref[...]

    inf = jnp.full((BQ, 1), jnp.inf, F32)
    bigc = jnp.full((BQ, 1), BIGF, F32)

    def colids(base):
        return (jax.lax.broadcasted_iota(jnp.int32, (1, CT), 1)
                + base).astype(F32)

    def chunk_min(d, colid, m, c):
        cm = jnp.min(d, axis=1, keepdims=True)
        ci = jnp.min(jnp.where(d == cm, colid, BIGF), axis=1, keepdims=True)
        take = cm < m
        return jnp.where(take, cm, m), jnp.where(take, ci, c)

    def abody(j, carry):
        m, c = carry
        base = pl.multiple_of(lo + j * CT, 128)
        bc = _dot(aq, bfull_ref[pl.ds(base, CT), :], ((1,), (1,)))
        same = batq == bcand_ref[0:1, pl.ds(base, CT)]
        d = jnp.where(same, bc, jnp.inf)
        d_scr[:, pl.ds(base, CT)] = d
        return chunk_min(d, colids(base), m, c)

    m0, c0 = jax.lax.fori_loop(0, nch, abody, (inf, bigc))

    def tbody(_, carry):
        m, c, mean, mx = carry
        w = jnp.exp(-10.0 * m)

        def jbody(j, icarry):
            m2, c2, acc = icarry
            base = pl.multiple_of(lo + j * CT, 128)
            chunk = d_scr[:, pl.ds(base, CT)]
            colid = colids(base)
            hit = colid == c
            acc = acc + _dot(hit.astype(F32), hm_ref[pl.ds(base, CT), :],
                             ((1,), (0,)))
            chunk = jnp.where(hit, jnp.inf, chunk)
            d_scr[:, pl.ds(base, CT)] = chunk
            m2, c2 = chunk_min(chunk, colid, m2, c2)
            return m2, c2, acc

        m2, c2, acc = jax.lax.fori_loop(
            0, nch, jbody, (inf, bigc, jnp.zeros((BQ, 16), F32)))
        msg = w * acc
        return m2, c2, mean + msg, jnp.maximum(mx, msg)

    _, _, mean, mx = jax.lax.fori_loop(
        0, K, tbody,
        (m0, c0, jnp.zeros((BQ, 16), F32), jnp.full((BQ, 16), -jnp.inf, F32)))

    out_ref[...] = (_dot(tq_ref[...], o1[...], ((1,), (0,)))
                    + _dot(mean * (1.0 / K), o2a[...], ((1,), (0,)))
                    + _dot(mx, o2b[...], ((1,), (0,)))
                    + bias[...])


def _final_body(hc_ref, br_ref, c1w, c1b, c2w, c2b, c3w, c3b, out_ref):
    gid = jax.lax.broadcasted_iota(jnp.int32, (N_GRAPHS, NPAD), 0)
    onehot = (gid == br_ref[...]).astype(F32)
    xs = _dot(onehot, hc_ref[...], ((1,), (0,)))
    cnt = jnp.sum(onehot, axis=1, keepdims=True)
    xp = xs / jnp.maximum(cnt, 1.0)
    z = jnp.maximum(_dot(xp, c1w[...], ((1,), (0,))) + c1b[...], 0.0)
    z = jnp.maximum(_dot(z, c2w[...], ((1,), (0,))) + c2b[...], 0.0)
    out_ref[...] = _dot(z, c3w[...], ((1,), (0,))) + c3b[...]


def _whole(shape):
    n = len(shape)
    return pl.BlockSpec(shape, lambda b: (0,) * n)


def _run_prep(xp, brow, bcol):
    return pl.pallas_call(
        _prep_body,
        out_shape=jax.ShapeDtypeStruct((NEXT, 8), F32),
    )(xp, brow, bcol)


def _run_dense(h, p, i):
    fin = h.shape[1]
    outs = (jax.ShapeDtypeStruct((NEXT, 16), F32),
            jax.ShapeDtypeStruct((NEXT, 8), F32),
            jax.ShapeDtypeStruct((NEXT, 8), F32),
            jax.ShapeDtypeStruct((NEXT, 16), F32))
    r2 = lambda v: v.reshape(1, -1)
    return pl.pallas_call(_dense_body, out_shape=outs)(
        h,
        p['ft%d_1_W' % i], r2(p['ft%d_1_b' % i]),
        p['ft%d_2_W' % i], r2(p['ft%d_2_b' % i]),
        p['ft%d_3_W' % i], r2(p['ft%d_3_b' % i]),
        p['gn%d_s_W' % i], r2(p['gn%d_s_b' % i]),
        p['gn%d_h_W' % i], r2(p['gn%d_h_b' % i]))


def _run_gravnet(lo, nch, A, Bs, t, hm, bcol, brow, o1, o2a, o2b, bias):
    grid = (NBLK,)
    in_specs = [
        pl.BlockSpec(memory_space=pltpu.SMEM),
        pl.BlockSpec(memory_space=pltpu.SMEM),
        pl.BlockSpec((BQ, 8), lambda b: (b, 0)),
        pl.BlockSpec((BQ, 1), lambda b: (b, 0)),
        pl.BlockSpec((BQ, 16), lambda b: (b, 0)),
        _whole((NEXT, 8)),
        _whole((1, NEXT)),
        _whole((NEXT, 16)),
        _whole((16, 16)), _whole((16, 16)), _whole((16, 16)),
        _whole((1, 16)),
    ]
    return pl.pallas_call(
        _gravnet_body,
        grid=grid,
        in_specs=in_specs,
        out_specs=pl.BlockSpec((BQ, 16), lambda b: (b, 0)),
        out_shape=jax.ShapeDtypeStruct((NPAD, 16), F32),
        scratch_shapes=[pltpu.VMEM((BQ, NEXT), F32)],
        compiler_params=pltpu.CompilerParams(
            dimension_semantics=("arbitrary",)),
    )(lo, nch, A, bcol, t, Bs, brow, hm, o1, o2a, o2b, bias)


def _run_final(hc, brow, p):
    r2 = lambda v: v.reshape(1, -1)
    return pl.pallas_call(
        _final_body,
        out_shape=jax.ShapeDtypeStruct((N_GRAPHS, 3), F32),
    )(hc, brow, p['c1_W'], r2(p['c1_b']), p['c2_W'], r2(p['c2_b']),
      p['c3_W'], r2(p['c3_b']))


@jax.jit
def kernel(x, edge_index, batch, params):
    del edge_index
    p = params
    pad = NEXT - N_NODES
    xp = jnp.pad(x, ((0, pad), (0, 0)))
    bpad = jnp.concatenate(
        [batch.astype(jnp.int32),
         jnp.full((pad,), N_GRAPHS, jnp.int32)])
    bcol = bpad[:, None]
    brow = bpad[None, :]

    offs = jnp.searchsorted(bpad, jnp.arange(N_GRAPHS + 2, dtype=jnp.int32),
                            side='left').astype(jnp.int32)
    qs = jnp.arange(NBLK, dtype=jnp.int32) * BQ
    g_lo = bpad[qs]
    g_hi = bpad[qs + BQ - 1]
    lo_raw = offs[g_lo]
    hi = offs[g_hi + 1]
    lo = (lo_raw // 128) * 128
    nch = (hi - lo + CT - 1) // CT

    h = _run_prep(xp, brow, bcol)
    feats = []
    for i in (1, 2, 3):
        t, A, Bs, hm = _run_dense(h, p, i)
        sc = p['bn%d_g' % i] / jnp.sqrt(1.0 + BN_EPS)
        o1 = p['gn%d_o1W' % i] * sc[None, :]
        o2 = p['gn%d_o2W' % i] * sc[None, :]
        bias = (p['gn%d_o2b' % i] * sc + p['bn%d_b' % i]).reshape(1, 16)
        hq = _run_gravnet(lo, nch, A, Bs, t, hm, bcol, brow,
                          o1, o2[:16], o2[16:], bias)
        feats.append(hq)
        h = jnp.pad(hq, ((0, CT), (0, 0)))
    hc = jnp.concatenate(feats, axis=1)
    return _run_final(hc, brow[:, :NPAD], p)


# speedup vs baseline: 1.1376x; 1.0030x over previous
import jax
import jax.numpy as jnp
from jax.experimental import pallas as pl
from jax.experimental.pallas import tpu as pltpu

N_NODES = 10000
N_GRAPHS = 8
K = 16
BN_EPS = 1e-5

NPAD = 10240
BQ = 256
CT = 1536
NEXT = NPAD + CT
NBLK = NPAD // BQ
BIGF = float(NEXT)
F32 = jnp.float32


def _dot(a, b, dims):
    return jax.lax.dot_general(a, b, (dims, ((), ())),
                               preferred_element_type=F32)


def _prep_body(x_ref, br_ref, bc_ref, h0_ref):
    x = x_ref[...]
    gid = jax.lax.broadcasted_iota(jnp.int32, (N_GRAPHS, NEXT), 0)
    onehot = (gid == br_ref[...]).astype(F32)
    gmsum = _dot(onehot, x, ((1,), (0,)))
    cnt = jnp.sum(onehot, axis=1, keepdims=True)
    gm = gmsum / jnp.maximum(cnt, 1.0)
    gidT = jax.lax.broadcasted_iota(jnp.int32, (NEXT, N_GRAPHS), 1)
    onehotT = (gidT == bc_ref[...]).astype(F32)
    gmb = _dot(onehotT, gm, ((1,), (0,)))
    h0_ref[...] = jnp.concatenate([x, gmb], axis=1)


def _elu(a):
    return jnp.where(a > 0, a, jnp.exp(a) - 1.0)


def _dense_body(h_ref, w1, b1, w2, b2, w3, b3, sw, sb, hw, hb,
                t_ref, a_ref, bside_ref, hm_ref):
    h = h_ref[...]
    a = _elu(_dot(h, w1[...], ((1,), (0,))) + b1[...])
    a = _elu(_dot(a, w2[...], ((1,), (0,))) + b2[...])
    t = jnp.tanh(_dot(a, w3[...], ((1,), (0,))) + b3[...])
    t_ref[...] = t
    s = _dot(t, sw[...], ((1,), (0,))) + sb[...]
    sq = jnp.sum(s * s, axis=1, keepdims=True)
    one = jnp.ones_like(sq)
    zero3 = jnp.zeros((NEXT, 3), F32)
    a_ref[...] = jnp.concatenate([-2.0 * s, sq, one, zero3], axis=1)
    bside_ref[...] = jnp.concatenate([s, one, sq, zero3], axis=1)
    hm_ref[...] = _dot(t, hw[...], ((1,), (0,))) + hb[...]


def _gravnet_body(lo_ref, nch_ref, aq_ref, bq_ref, tq_ref, bfull_ref,
                  bcand_ref, hm_ref, o1, o2a, o2b, bias, out_ref, d_scr):
    b = pl.program_id(0)
    lo = lo_ref[b]
    nch = nch_ref[b]
    aq = aq_ref[...]
    batq = bq_ref[...]

    inf = jnp.full((BQ, 1), jnp.inf, F32)
    bigc = jnp.full((BQ, 1), BIGF, F32)

    def colids(base):
        return (jax.lax.broadcasted_iota(jnp.int32, (1, CT), 1)
                + base).astype(F32)

    def chunk_min(d, colid, m, c):
        cm = jnp.min(d, axis=1, keepdims=True)
        ci = jnp.min(jnp.where(d == cm, colid, BIGF), axis=1, keepdims=True)
        take = cm < m
        return jnp.where(take, cm, m), jnp.where(take, ci, c)

    def abody(j, carry):
        m, c = carry
        base = pl.multiple_of(lo + j * CT, 128)
        bc = _dot(aq, bfull_ref[pl.ds(base, CT), :], ((1,), (1,)))
        same = batq == bcand_ref[0:1, pl.ds(base, CT)]
        d = jnp.where(same, bc, jnp.inf)
        d_scr[:, pl.ds(base, CT)] = d
        return chunk_min(d, colids(base), m, c)

    m0, c0 = jax.lax.fori_loop(0, nch, abody, (inf, bigc))

    def tbody(_, carry):
        m, c, mean, mx = carry
        w = jnp.exp(-10.0 * m)

        def jbody(j, icarry):
            m2, c2, acc = icarry
            base = pl.multiple_of(lo + j * CT, 128)
            chunk = d_scr[:, pl.ds(base, CT)]
            colid = colids(base)
            hit = colid == c
            acc = acc + _dot(hit.astype(F32), hm_ref[pl.ds(base, CT), :],
                             ((1,), (0,)))
            chunk = jnp.where(hit, jnp.inf, chunk)
            d_scr[:, pl.ds(base, CT)] = chunk
            m2, c2 = chunk_min(chunk, colid, m2, c2)
            return m2, c2, acc

        m2, c2, acc = jax.lax.fori_loop(
            0, nch, jbody, (inf, bigc, jnp.zeros((BQ, 16), F32)))
        msg = w * acc
        return m2, c2, mean + msg, jnp.maximum(mx, msg)

    _, _, mean, mx = jax.lax.fori_loop(
        0, K, tbody,
        (m0, c0, jnp.zeros((BQ, 16), F32), jnp.full((BQ, 16), -jnp.inf, F32)))

    out_ref[...] = (_dot(tq_ref[...], o1[...], ((1,), (0,)))
                    + _dot(mean * (1.0 / K), o2a[...], ((1,), (0,)))
                    + _dot(mx, o2b[...], ((1,), (0,)))
                    + bias[...])


def _final_body(hc_ref, br_ref, c1w, c1b, c2w, c2b, c3w, c3b, out_ref):
    gid = jax.lax.broadcasted_iota(jnp.int32, (N_GRAPHS, NPAD), 0)
    onehot = (gid == br_ref[...]).astype(F32)
    xs = _dot(onehot, hc_ref[...], ((1,), (0,)))
    cnt = jnp.sum(onehot, axis=1, keepdims=True)
    xp = xs / jnp.maximum(cnt, 1.0)
    z = jnp.maximum(_dot(xp, c1w[...], ((1,), (0,))) + c1b[...], 0.0)
    z = jnp.maximum(_dot(z, c2w[...], ((1,), (0,))) + c2b[...], 0.0)
    out_ref[...] = _dot(z, c3w[...], ((1,), (0,))) + c3b[...]


def _whole(shape):
    n = len(shape)
    return pl.BlockSpec(shape, lambda b: (0,) * n)


def _run_prep(xp, brow, bcol):
    return pl.pallas_call(
        _prep_body,
        out_shape=jax.ShapeDtypeStruct((NEXT, 8), F32),
    )(xp, brow, bcol)


def _run_dense(h, p, i):
    outs = (jax.ShapeDtypeStruct((NEXT, 16), F32),
            jax.ShapeDtypeStruct((NEXT, 8), F32),
            jax.ShapeDtypeStruct((NEXT, 8), F32),
            jax.ShapeDtypeStruct((NEXT, 16), F32))
    r2 = lambda v: v.reshape(1, -1)
    return pl.pallas_call(_dense_body, out_shape=outs)(
        h,
        p['ft%d_1_W' % i], r2(p['ft%d_1_b' % i]),
        p['ft%d_2_W' % i], r2(p['ft%d_2_b' % i]),
        p['ft%d_3_W' % i], r2(p['ft%d_3_b' % i]),
        p['gn%d_s_W' % i], r2(p['gn%d_s_b' % i]),
        p['gn%d_h_W' % i], r2(p['gn%d_h_b' % i]))


def _run_gravnet(lo, nch, A, Bs, t, hm, bcol, brow, o1, o2a, o2b, bias):
    grid = (NBLK,)
    in_specs = [
        pl.BlockSpec(memory_space=pltpu.SMEM),
        pl.BlockSpec(memory_space=pltpu.SMEM),
        pl.BlockSpec((BQ, 8), lambda b: (b, 0)),
        pl.BlockSpec((BQ, 1), lambda b: (b, 0)),
        pl.BlockSpec((BQ, 16), lambda b: (b, 0)),
        _whole((NEXT, 8)),
        _whole((1, NEXT)),
        _whole((NEXT, 16)),
        _whole((16, 16)), _whole((16, 16)), _whole((16, 16)),
        _whole((1, 16)),
    ]
    return pl.pallas_call(
        _gravnet_body,
        grid=grid,
        in_specs=in_specs,
        out_specs=pl.BlockSpec((BQ, 16), lambda b: (b, 0)),
        out_shape=jax.ShapeDtypeStruct((NPAD, 16), F32),
        scratch_shapes=[pltpu.VMEM((BQ, NEXT), F32)],
        compiler_params=pltpu.CompilerParams(
            dimension_semantics=("arbitrary",)),
    )(lo, nch, A, bcol, t, Bs, brow, hm, o1, o2a, o2b, bias)


def _run_final(hc, brow, p):
    r2 = lambda v: v.reshape(1, -1)
    return pl.pallas_call(
        _final_body,
        out_shape=jax.ShapeDtypeStruct((N_GRAPHS, 3), F32),
    )(hc, brow, p['c1_W'], r2(p['c1_b']), p['c2_W'], r2(p['c2_b']),
      p['c3_W'], r2(p['c3_b']))


@jax.jit
def kernel(x, edge_index, batch, params):
    del edge_index
    p = params
    pad = NEXT - N_NODES
    xp = jnp.pad(x, ((0, pad), (0, 0)))
    bpad = jnp.concatenate(
        [batch.astype(jnp.int32),
         jnp.full((pad,), N_GRAPHS, jnp.int32)])
    bcol = bpad[:, None]
    brow = bpad[None, :]

    offs = jnp.searchsorted(bpad, jnp.arange(N_GRAPHS + 2, dtype=jnp.int32),
                            side='left').astype(jnp.int32)
    qs = jnp.arange(NBLK, dtype=jnp.int32) * BQ
    g_lo = bpad[qs]
    g_hi = bpad[qs + BQ - 1]
    lo_raw = offs[g_lo]
    hi = offs[g_hi + 1]
    lo = (lo_raw // 128) * 128
    nch = (hi - lo + CT - 1) // CT

    h = _run_prep(xp, brow, bcol)
    feats = []
    for i in (1, 2, 3):
        t, A, Bs, hm = _run_dense(h, p, i)
        sc = p['bn%d_g' % i] / jnp.sqrt(1.0 + BN_EPS)
        o1 = p['gn%d_o1W' % i] * sc[None, :]
        o2 = p['gn%d_o2W' % i] * sc[None, :]
        bias = (p['gn%d_o2b' % i] * sc + p['bn%d_b' % i]).reshape(1, 16)
        hq = _run_gravnet(lo, nch, A, Bs, t, hm, bcol, brow,
                          o1, o2[:16], o2[16:], bias)
        feats.append(hq)
        h = jnp.pad(hq, ((0, CT), (0, 0)))
    hc = jnp.concatenate(feats, axis=1)
    return _run_final(hc, brow[:, :NPAD], p)
